# baseline (speedup 1.0000x reference)
import functools

import jax
import jax.numpy as jnp
from jax import lax
from jax.experimental import pallas as pl
from jax.experimental.pallas import tpu as pltpu
from jax.experimental.pallas import tpu_sc as plsc

D = 32
DD = D * D
B = 16384
L = 16
NC, NS = 2, 16
NW = NC * NS
BPW = B // NW
K = 32
NCHUNK = BPW // K
QC = 4
NQ = NCHUNK // QC
QE = QC * K


def _sc_call(ent_t, att, W_att):
    mesh = plsc.VectorSubcoreMesh(core_axis_name="c", subcore_axis_name="s")

    @functools.partial(
        pl.kernel,
        mesh=mesh,
        compiler_params=pltpu.CompilerParams(needs_layout_passes=False),
        out_type=jax.ShapeDtypeStruct((D, B), jnp.float32),
        scratch_types=[
            pltpu.VMEM((BPW,), jnp.int32),
            pltpu.VMEM((2, D, QE), jnp.float32),
            pltpu.VMEM((2, K, DD), jnp.float32),
            pltpu.VMEM((2, D, QE), jnp.float32),
            pltpu.SemaphoreType.DMA,
            pltpu.SemaphoreType.DMA,
            pltpu.SemaphoreType.DMA,
            pltpu.SemaphoreType.DMA,
            pltpu.SemaphoreType.DMA,
            pltpu.SemaphoreType.DMA,
        ],
    )
    def body(ent_hbm, att_hbm, w_hbm, out_hbm, idx_v, entq_v, rows_v, outq_v,
             semr0, semr1, seme0, seme1, semo0, semo1):
        wid = lax.axis_index("s") * NC + lax.axis_index("c")
        base = wid * BPW
        pltpu.sync_copy(att_hbm.at[pl.ds(base, BPW)], idx_v)

        semr = (semr0, semr1)
        seme = (seme0, seme1)
        semo = (semo0, semo1)

        def rows_desc(c, par):
            return pltpu.make_async_copy(
                w_hbm.at[idx_v.at[pl.ds(c * K, K)]], rows_v.at[par], semr[par])

        def ent_desc(q, qpar):
            return pltpu.make_async_copy(
                ent_hbm.at[:, pl.ds(base + q * QE, QE)], entq_v.at[qpar],
                seme[qpar])

        def out_desc(q, qpar):
            return pltpu.make_async_copy(
                outq_v.at[qpar], out_hbm.at[:, pl.ds(base + q * QE, QE)],
                semo[qpar])

        lane = jax.lax.iota(jnp.int32, 16)

        def compute(par, qpar, eq_base):
            def elem(e, _):
                eq = jnp.zeros((L,), jnp.int32) + (eq_base + e)
                ev0 = plsc.load_gather(entq_v.at[qpar], [lane, eq])
                ev1 = plsc.load_gather(entq_v.at[qpar], [lane + L, eq])
                a0 = jnp.zeros((L,), jnp.float32)
                a1 = jnp.zeros((L,), jnp.float32)
                b0 = jnp.zeros((L,), jnp.float32)
                b1 = jnp.zeros((L,), jnp.float32)
                for i in range(0, D, 2):
                    ev = ev0 if i < L else ev1
                    s0 = ev[i % L]
                    s1 = ev[(i + 1) % L]
                    a0 = a0 + s0 * rows_v[par, e, pl.ds(i * D, L)]
                    a1 = a1 + s0 * rows_v[par, e, pl.ds(i * D + L, L)]
                    b0 = b0 + s1 * rows_v[par, e, pl.ds(i * D + D, L)]
                    b1 = b1 + s1 * rows_v[par, e, pl.ds(i * D + D + L, L)]
                plsc.store_scatter(outq_v.at[qpar], [lane, eq], a0 + b0)
                plsc.store_scatter(outq_v.at[qpar], [lane + L, eq], a1 + b1)
                return 0

            lax.fori_loop(0, K, elem, 0, unroll=2)

        ent_desc(0, 0).start()
        ent_desc(1, 1).start()
        rows_desc(0, 0).start()
        rows_desc(1, 1).start()

        for q in range(NQ):
            qpar = q % 2
            ent_desc(q, qpar).wait()
            if q >= 2:
                out_desc(q - 2, qpar).wait()

            def pair(cp, _, q=q, qpar=qpar):
                for par in range(2):
                    cq = cp * 2 + par
                    c = q * QC + cq
                    rows_desc(c, par).wait()
                    compute(par, qpar, cq * K)

                    @pl.when(c + 2 < NCHUNK)
                    def _():
                        rows_desc(c + 2, par).start()
                return 0

            lax.fori_loop(0, QC // 2, pair, 0)

            if q + 2 < NQ:
                ent_desc(q + 2, qpar).start()

            out_desc(q, qpar).start()

        out_desc(NQ - 2, 0).wait()
        out_desc(NQ - 1, 1).wait()

    return body(ent_t, att, W_att)


def kernel(ent, att, W_att):
    out_t = _sc_call(ent.T, att.astype(jnp.int32), W_att)
    return out_t.T
